# baseline (speedup 1.0000x reference)
import functools

import jax
import jax.numpy as jnp
from jax import lax
from jax.experimental import pallas as pl
from jax.experimental.pallas import tpu as pltpu
from jax.experimental.pallas import tpu_sc as plsc

_B = 1024
_H = 500
_HP = 512
_K = 400.0
_N = 500.0
_EPS = 0.1
_ITERS = 8
_PAD_Q = 1e30

_NW = 32
_RPW = _B // _NW

_NT = (((1,), (1,)), ((), ()))


def _dot_nt(a, b):
    return lax.dot_general(a, b, _NT, preferred_element_type=jnp.float32)


def _s_to_q(s):
    m = jnp.max(jnp.maximum(s, jnp.abs(s - 1.0)))
    a = 1.0 / (_EPS * m * m)
    q = jnp.exp((2.0 * s - 1.0) * a)
    pad = jnp.full((_B, _HP - _H), _PAD_Q, jnp.float32)
    return jnp.concatenate([q, pad], axis=1)


def _pad_s(s):
    return jnp.concatenate([s, jnp.zeros((_B, _HP - _H), jnp.float32)], axis=1)


def _mask_mul(s_p, q_p, xr):
    xc = xr[:, :1]
    h = s_p * (1.0 - xc / (q_p + xc))
    return h[:, :_H]


def _tc_first(x_ref, w_ref, b_ref, s_ref, q_ref):
    s = jnp.maximum(_dot_nt(x_ref[...], w_ref[...]) + b_ref[...], 0.0)
    s_ref[...] = _pad_s(s)
    q_ref[...] = _s_to_q(s)


def _tc_mid(s_ref, q_ref, xr_ref, w_ref, b_ref, s2_ref, q2_ref):
    h = _mask_mul(s_ref[...], q_ref[...], xr_ref[...])
    s = jnp.maximum(_dot_nt(h, w_ref[...]) + b_ref[...], 0.0)
    s2_ref[...] = _pad_s(s)
    q2_ref[...] = _s_to_q(s)


def _tc_last(s_ref, q_ref, xr_ref, w_ref, b_ref, o_ref):
    h = _mask_mul(s_ref[...], q_ref[...], xr_ref[...])
    o_ref[...] = _dot_nt(h, w_ref[...]) + b_ref[...]


@functools.partial(
    pl.kernel,
    out_type=jax.ShapeDtypeStruct((_B, 16), jnp.float32),
    mesh=plsc.VectorSubcoreMesh(core_axis_name="c", subcore_axis_name="s"),
    compiler_params=pltpu.CompilerParams(needs_layout_passes=False),
    scratch_types=[
        pltpu.VMEM((_RPW, _HP), jnp.float32),
        pltpu.VMEM((_RPW, 16), jnp.float32),
    ],
)
def _sc_newton(q_hbm, x_hbm, q_v, x_v):
    wid = lax.axis_index("s") * 2 + lax.axis_index("c")
    base = wid * _RPW
    pltpu.sync_copy(q_hbm.at[pl.ds(base, _RPW)], q_v)

    def row_body(r, carry):
        def newton(_, x):
            s1v = jnp.zeros((16,), jnp.float32)
            s2v = jnp.zeros((16,), jnp.float32)
            for j in range(_HP // 16):
                t = 1.0 / (q_v[r, pl.ds(j * 16, 16)] + x)
                s1v = s1v + t
                s2v = s2v + t * t
            s1 = jnp.sum(s1v)
            s2 = jnp.sum(s2v)
            xn = x - (x * s1 - (_N - _K)) / (s1 - x * s2)
            return jnp.abs(xn)

        x = lax.fori_loop(0, _ITERS, newton,
                          jnp.full((16,), 1e-6, jnp.float32))
        x_v[r] = x
        return carry

    lax.fori_loop(0, _RPW, row_body, jnp.int32(0))
    pltpu.sync_copy(x_v, x_hbm.at[pl.ds(base, _RPW)])


def _tc_call(body, out_shapes, *args):
    return pl.pallas_call(
        body,
        out_shape=out_shapes,
    )(*args)


@jax.jit
def kernel(x, W1, b1, W2, b2, W3, b3, W4, b4):
    f32 = jnp.float32
    sq = [jax.ShapeDtypeStruct((_B, _HP), f32)] * 2
    s1, q1 = _tc_call(_tc_first, sq, x, W1, b1.reshape(1, -1))
    x1 = _sc_newton(q1)
    s2, q2 = _tc_call(_tc_mid, sq, s1, q1, x1, W2, b2.reshape(1, -1))
    x2 = _sc_newton(q2)
    s3, q3 = _tc_call(_tc_mid, sq, s2, q2, x2, W3, b3.reshape(1, -1))
    x3 = _sc_newton(q3)
    out = _tc_call(_tc_last, jax.ShapeDtypeStruct((_B, W4.shape[0]), f32),
                   s3, q3, x3, W4, b4.reshape(1, -1))
    return out


# speedup vs baseline: 2.6417x; 2.6417x over previous
import functools

import jax
import jax.numpy as jnp
from jax.experimental import pallas as pl
from jax.experimental.pallas import tpu as pltpu

_B = 1024
_K = 400.0
_N = 500.0
_EPS = 0.1
_ITERS = 10

_NT = (((1,), (1,)), ((), ()))


def _soft_topk_mul(s):
    m = jnp.max(jnp.maximum(s, jnp.abs(s - 1.0)))
    a = 1.0 / (_EPS * m * m)
    q = jnp.exp((2.0 * s - 1.0) * a)

    def body(_, carry):
        x, xp, fp = carry
        f = x * jnp.sum(1.0 / (q + x), axis=1, keepdims=True)
        df = f - fp
        xn = jnp.where(jnp.abs(df) > 1e-3,
                       x - (f - (_N - _K)) * (x - xp) / df, x)
        return jnp.abs(xn), x, f

    zero = s[:, :1] * 0.0
    x, _, _ = jax.lax.fori_loop(
        0, _ITERS, body, (zero + 1e-6, zero, zero))
    mask = 1.0 - x / (q + x)
    return s * mask


def _dot_nt(a, b):
    return jax.lax.dot_general(a, b, _NT, preferred_element_type=jnp.float32)


def _fwd(x_ref, w1_ref, b1_ref, w2_ref, b2_ref, w3_ref, b3_ref, w4_ref,
         b4_ref, o_ref):
    s = jnp.maximum(_dot_nt(x_ref[...], w1_ref[...]) + b1_ref[...], 0.0)
    for w_ref, b_ref in ((w2_ref, b2_ref), (w3_ref, b3_ref)):
        h = _soft_topk_mul(s)
        s = jnp.maximum(_dot_nt(h, w_ref[...]) + b_ref[...], 0.0)
    h = _soft_topk_mul(s)
    o_ref[...] = _dot_nt(h, w4_ref[...]) + b4_ref[...]


@jax.jit
def kernel(x, W1, b1, W2, b2, W3, b3, W4, b4):
    return pl.pallas_call(
        _fwd,
        out_shape=jax.ShapeDtypeStruct((_B, W4.shape[0]), jnp.float32),
    )(x, W1, b1.reshape(1, -1), W2, b2.reshape(1, -1), W3, b3.reshape(1, -1),
      W4, b4.reshape(1, -1))


# speedup vs baseline: 4.3388x; 1.6424x over previous
import functools

import jax
import jax.numpy as jnp
from jax.experimental import pallas as pl
from jax.experimental.pallas import tpu as pltpu

_B = 1024
_K = 400.0
_N = 500.0
_EPS = 0.1
_ITERS = 6

_NT = (((1,), (1,)), ((), ()))


def _soft_topk_mul(s):
    m = jnp.max(jnp.maximum(s, jnp.abs(s - 1.0)))
    a = 1.0 / (_EPS * m * m)
    q = jnp.exp((2.0 * s - 1.0) * a)

    def body(_, x):
        t = 1.0 / (q + x)
        s1 = jnp.sum(t, axis=1, keepdims=True)
        s2 = jnp.sum(t * t, axis=1, keepdims=True)
        xn = x - (x * s1 - (_N - _K)) / (s1 - x * s2)
        return jnp.abs(xn)

    x = jax.lax.fori_loop(0, _ITERS, body,
                          jnp.full((_B, 1), 1e-6, jnp.float32))
    mask = 1.0 - x / (q + x)
    return s * mask


def _dot_nt(a, b):
    return jax.lax.dot_general(a, b, _NT, preferred_element_type=jnp.float32)


def _fwd(x_ref, w1_ref, b1_ref, w2_ref, b2_ref, w3_ref, b3_ref, w4_ref,
         b4_ref, o_ref):
    s = jnp.maximum(_dot_nt(x_ref[...], w1_ref[...]) + b1_ref[...], 0.0)
    for w_ref, b_ref in ((w2_ref, b2_ref), (w3_ref, b3_ref)):
        h = _soft_topk_mul(s)
        s = jnp.maximum(_dot_nt(h, w_ref[...]) + b_ref[...], 0.0)
    h = _soft_topk_mul(s)
    o_ref[...] = _dot_nt(h, w4_ref[...]) + b4_ref[...]


@jax.jit
def kernel(x, W1, b1, W2, b2, W3, b3, W4, b4):
    return pl.pallas_call(
        _fwd,
        out_shape=jax.ShapeDtypeStruct((_B, W4.shape[0]), jnp.float32),
    )(x, W1, b1.reshape(1, -1), W2, b2.reshape(1, -1), W3, b3.reshape(1, -1),
      W4, b4.reshape(1, -1))


# speedup vs baseline: 4.5206x; 1.0419x over previous
import functools

import jax
import jax.numpy as jnp
from jax.experimental import pallas as pl
from jax.experimental.pallas import tpu as pltpu

_B = 1024
_K = 400.0
_N = 500.0
_EPS = 0.1
_ITERS = 5

_NT = (((1,), (1,)), ((), ()))


def _soft_topk_mul(s):
    m = jnp.max(jnp.maximum(s, jnp.abs(s - 1.0)))
    a = 1.0 / (_EPS * m * m)
    q = jnp.exp((2.0 * s - 1.0) * a)

    def body(_, x):
        t = 1.0 / (q + x)
        s1 = jnp.sum(t, axis=1, keepdims=True)
        s2 = jnp.sum(t * t, axis=1, keepdims=True)
        xn = x - (x * s1 - (_N - _K)) / (s1 - x * s2)
        return jnp.abs(xn)

    x = jax.lax.fori_loop(0, _ITERS, body,
                          jnp.full((_B, 1), 1e-6, jnp.float32))
    t = 1.0 / (q + x)
    s1 = jnp.sum(t, axis=1, keepdims=True)
    s2 = jnp.sum(t * t, axis=1, keepdims=True)
    x = jnp.abs(x - (x * s1 - (_N - _K)) / (s1 - x * s2))
    return s - (s * x) * t


def _dot_nt(a, b):
    return jax.lax.dot_general(a, b, _NT, preferred_element_type=jnp.float32)


def _fwd(x_ref, w1_ref, b1_ref, w2_ref, b2_ref, w3_ref, b3_ref, w4_ref,
         b4_ref, o_ref):
    s = jnp.maximum(_dot_nt(x_ref[...], w1_ref[...]) + b1_ref[...], 0.0)
    for w_ref, b_ref in ((w2_ref, b2_ref), (w3_ref, b3_ref)):
        h = _soft_topk_mul(s)
        s = jnp.maximum(_dot_nt(h, w_ref[...]) + b_ref[...], 0.0)
    h = _soft_topk_mul(s)
    o_ref[...] = _dot_nt(h, w4_ref[...]) + b4_ref[...]


@jax.jit
def kernel(x, W1, b1, W2, b2, W3, b3, W4, b4):
    return pl.pallas_call(
        _fwd,
        out_shape=jax.ShapeDtypeStruct((_B, W4.shape[0]), jnp.float32),
    )(x, W1, b1.reshape(1, -1), W2, b2.reshape(1, -1), W3, b3.reshape(1, -1),
      W4, b4.reshape(1, -1))
